# baseline (speedup 1.0000x reference)
import functools

import jax
import jax.numpy as jnp
from jax import lax
from jax.experimental import pallas as pl
from jax.experimental.pallas import tpu as pltpu

N_NODES_C = 50000
N_EDGES_C = 800000
E_PAD = 819200
N_PAD = 50048
NG = 16
EPS = 1e-8

BN = 2000
BE = 4096


def _fullspec(shape):
    return pl.BlockSpec(shape, lambda i: tuple(0 for _ in shape))


def _r2(b):
    return b.reshape(1, -1)


def _sigmoid(x):
    return jax.nn.sigmoid(x)


def _dot(a, b):
    return jax.lax.dot_general(a, b, (((1,), (0,)), ((), ())),
                               preferred_element_type=jnp.float32)


def _embed_nodes_body(ns, nv, lnb, whT, wswT, wsb, wvT, wsvT, wsvb, out):
    s_ln = jnp.zeros_like(ns[...]) + lnb[...]
    v = nv[...]
    nsq = jnp.maximum(jnp.sum(v * v, axis=1, keepdims=True), EPS)
    v = v / jnp.sqrt(nsq)
    vh = [_dot(v[:, x:x + 1], whT[...]) for x in range(3)]
    vn = jnp.sqrt(jnp.maximum(vh[0] ** 2 + vh[1] ** 2 + vh[2] ** 2, EPS))
    s = _dot(jnp.concatenate([s_ln, vn], axis=1), wswT[...]) + wsb[...]
    gate = _sigmoid(_dot(s, wsvT[...]) + wsvb[...])
    vo = [_dot(vh[x], wvT[...]) * gate for x in range(3)]
    z = jnp.zeros_like(s[:, :2])
    out[...] = jnp.concatenate([s] + vo + [z], axis=1)


def _embed_edges_body(es0, ev0, lnb, whT, wswT, wsb, wvT, wsvT, wsvb, out):
    s_ln = jnp.zeros_like(es0[...]) + lnb[...]
    v = ev0[...]
    nsq = jnp.maximum(jnp.sum(v * v, axis=1, keepdims=True), EPS)
    v = v / jnp.sqrt(nsq)
    vh = [_dot(v[:, x:x + 1], whT[...]) for x in range(3)]
    vn = jnp.sqrt(jnp.maximum(vh[0] ** 2 + vh[1] ** 2 + vh[2] ** 2, EPS))
    s = _dot(jnp.concatenate([s_ln, vn], axis=1), wswT[...]) + wsb[...]
    gate = _sigmoid(_dot(s, wsvT[...]) + wsvb[...])
    vo = [vh[x] * wvT[...] * gate for x in range(3)]
    z = jnp.zeros_like(gate)
    out[...] = jnp.concatenate([s] + vo + [z], axis=1)


def _gvp_block(s_in, v_in, whT, wswT, wsb, wvT, wsvT, wsvb, relu):
    vh = [_dot(v_in[x], whT) for x in range(3)]
    vn = jnp.sqrt(jnp.maximum(vh[0] ** 2 + vh[1] ** 2 + vh[2] ** 2, EPS))
    s = _dot(jnp.concatenate([s_in, vn], axis=1), wswT) + wsb
    gate = _sigmoid(_dot(s, wsvT) + wsvb)
    v = [_dot(vh[x], wvT) * gate for x in range(3)]
    if relu:
        s = jnp.maximum(s, 0.0)
    return s, v


def _messages_body(src, dst, et, *ws):
    (m0whT, m0wswT, m0wsb, m0wvT, m0wsvT, m0wsvb,
     m1whT, m1wswT, m1wsb, m1wvT, m1wsvT, m1wsvb,
     m2whT, m2wswT, m2wsb, m2wvT, m2wsvT, m2wsvb, out) = ws
    S = src[...]
    D = dst[...]
    ET = et[...]
    s_in = jnp.concatenate([S[:, :8], ET[:, :4], D[:, :8]], axis=1)
    v_in = [jnp.concatenate([S[:, 8 + 2 * x:10 + 2 * x],
                             ET[:, 4 + x:5 + x],
                             D[:, 8 + 2 * x:10 + 2 * x]], axis=1)
            for x in range(3)]
    s, v = _gvp_block(s_in, v_in, m0whT[...], m0wswT[...], m0wsb[...],
                      m0wvT[...], m0wsvT[...], m0wsvb[...], True)
    s, v = _gvp_block(s, v, m1whT[...], m1wswT[...], m1wsb[...],
                      m1wvT[...], m1wsvT[...], m1wsvb[...], True)
    s, v = _gvp_block(s, v, m2whT[...], m2wswT[...], m2wsb[...],
                      m2wvT[...], m2wsvT[...], m2wsvb[...], False)
    one = jnp.ones_like(s[:, :1])
    z = jnp.zeros_like(one)
    out[...] = jnp.concatenate([s] + v + [one, z], axis=1)


def _tuple_ln(s, v, w, b):
    mu = jnp.mean(s, axis=1, keepdims=True)
    var = jnp.mean((s - mu) ** 2, axis=1, keepdims=True)
    sn = (s - mu) / jnp.sqrt(var + 1e-5) * w + b
    nsq = jnp.maximum(v[0] ** 2 + v[1] ** 2 + v[2] ** 2, EPS)
    vnorm = jnp.sqrt(jnp.mean(nsq, axis=1, keepdims=True))
    return sn, [v[x] / vnorm for x in range(3)]


def _node_update_body(xin, a0, a1, *ws):
    (n0w, n0b,
     f0whT, f0wswT, f0wsb, f0wvT, f0wsvT, f0wsvb,
     f1whT, f1wswT, f1wsb, f1wvT, f1wsvT, f1wsvb,
     n1w, n1b, out) = ws
    X = xin[...]
    A = a0[...] + a1[...]
    cnt = jnp.maximum(A[:, 14:15], 1.0)
    s = X[:, :8] + A[:, :8] / cnt
    v = [X[:, 8 + 2 * x:10 + 2 * x] + A[:, 8 + 2 * x:10 + 2 * x] / cnt
         for x in range(3)]
    s, v = _tuple_ln(s, v, n0w[...], n0b[...])
    fs, fv = _gvp_block(s, v, f0whT[...], f0wswT[...], f0wsb[...],
                        f0wvT[...], f0wsvT[...], f0wsvb[...], True)
    fs, fv = _gvp_block(fs, fv, f1whT[...], f1wswT[...], f1wsb[...],
                        f1wvT[...], f1wsvT[...], f1wsvb[...], False)
    s = s + fs
    v = [v[x] + fv[x] for x in range(3)]
    s, v = _tuple_ln(s, v, n1w[...], n1b[...])
    z = jnp.zeros_like(s[:, :2])
    out[...] = jnp.concatenate([s] + v + [z], axis=1)


def _out_pool_body(xin, bat, owhT, owswT, owsb, out, acc_s, acc_c):
    i = pl.program_id(0)
    X = xin[...]
    s = X[:, :8]
    v = [X[:, 8 + 2 * x:10 + 2 * x] for x in range(3)]
    vh = [_dot(v[x], owhT[...]) for x in range(3)]
    vn = jnp.sqrt(jnp.maximum(vh[0] ** 2 + vh[1] ** 2 + vh[2] ** 2, EPS))
    o = _dot(jnp.concatenate([s, vn], axis=1), owswT[...]) + owsb[...]
    bb = bat[0, 0, :]
    gid = jax.lax.broadcasted_iota(jnp.int32, (1, NG), 1)
    onehot = (bb[:, None] == gid).astype(jnp.float32)

    @pl.when(i == 0)
    def _():
        acc_s[...] = jnp.zeros_like(acc_s)
        acc_c[...] = jnp.zeros_like(acc_c)

    acc_s[...] += jnp.sum(onehot * o, axis=0, keepdims=True)
    acc_c[...] += jnp.sum(onehot, axis=0, keepdims=True)

    @pl.when(i == pl.num_programs(0) - 1)
    def _():
        out[...] = acc_s[...] / jnp.maximum(acc_c[...], 1.0)


def _embed_nodes(node_s, node_v, p):
    n = node_s.shape[0]
    ws = (_r2(p['lnb']), p['whT'], p['wswT'], _r2(p['wsb']),
          p['wvT'], p['wsvT'], _r2(p['wsvb']))
    grid = n // BN
    specs = [pl.BlockSpec((BN, 1), lambda i: (i, 0)),
             pl.BlockSpec((BN, 3), lambda i: (i, 0))] + \
            [_fullspec(w.shape) for w in ws]
    return pl.pallas_call(
        _embed_nodes_body,
        grid=(grid,),
        in_specs=specs,
        out_specs=pl.BlockSpec((BN, 16), lambda i: (i, 0)),
        out_shape=jax.ShapeDtypeStruct((n, 16), jnp.float32),
    )(node_s, node_v, *ws)


def _embed_edges(edge_s, edge_v, p):
    n = edge_s.shape[0]
    ws = (_r2(p['lnb']), p['whT'], p['wswT'], _r2(p['wsb']),
          p['wvT'], p['wsvT'], _r2(p['wsvb']))
    grid = n // BE
    specs = [pl.BlockSpec((BE, 1), lambda i: (i, 0)),
             pl.BlockSpec((BE, 3), lambda i: (i, 0))] + \
            [_fullspec(w.shape) for w in ws]
    return pl.pallas_call(
        _embed_edges_body,
        grid=(grid,),
        in_specs=specs,
        out_specs=pl.BlockSpec((BE, 8), lambda i: (i, 0)),
        out_shape=jax.ShapeDtypeStruct((n, 8), jnp.float32),
    )(edge_s, edge_v, *ws)


def _messages(src_rows, dst_rows, et, mp):
    n = src_rows.shape[0]
    ws = []
    for m in ('m0', 'm1', 'm2'):
        q = mp[m]
        ws += [q['whT'], q['wswT'], _r2(q['wsb']), q['wvT'], q['wsvT'],
               _r2(q['wsvb'])]
    grid = n // BE
    specs = [pl.BlockSpec((BE, 16), lambda i: (i, 0)),
             pl.BlockSpec((BE, 16), lambda i: (i, 0)),
             pl.BlockSpec((BE, 8), lambda i: (i, 0))] + \
            [_fullspec(w.shape) for w in ws]
    return pl.pallas_call(
        _messages_body,
        grid=(grid,),
        in_specs=specs,
        out_specs=pl.BlockSpec((BE, 16), lambda i: (i, 0)),
        out_shape=jax.ShapeDtypeStruct((n, 16), jnp.float32),
    )(src_rows, dst_rows, et, *ws)


def _node_update(x, a0, a1, lp):
    n = x.shape[0]
    ws = [_r2(lp['n0w']), _r2(lp['n0b'])]
    for m in ('ff0', 'ff1'):
        q = lp[m]
        ws += [q['whT'], q['wswT'], _r2(q['wsb']), q['wvT'], q['wsvT'],
               _r2(q['wsvb'])]
    ws += [_r2(lp['n1w']), _r2(lp['n1b'])]
    grid = n // BN
    specs = [pl.BlockSpec((BN, 16), lambda i: (i, 0))] * 3 + \
            [_fullspec(w.shape) for w in ws]
    return pl.pallas_call(
        _node_update_body,
        grid=(grid,),
        in_specs=specs,
        out_specs=pl.BlockSpec((BN, 16), lambda i: (i, 0)),
        out_shape=jax.ShapeDtypeStruct((n, 16), jnp.float32),
    )(x, a0, a1, *ws)


def _out_pool(x, batch3, op):
    n = x.shape[0]
    ws = (op['whT'], op['wswT'], _r2(op['wsb']))
    grid = n // BN
    specs = [pl.BlockSpec((BN, 16), lambda i: (i, 0)),
             pl.BlockSpec((1, 1, BN), lambda i: (i, 0, 0))] + \
            [_fullspec(w.shape) for w in ws]
    return pl.pallas_call(
        _out_pool_body,
        grid=(grid,),
        in_specs=specs,
        out_specs=_fullspec((1, NG)),
        out_shape=jax.ShapeDtypeStruct((1, NG), jnp.float32),
        scratch_shapes=[pltpu.VMEM((1, NG), jnp.float32),
                        pltpu.VMEM((1, NG), jnp.float32)],
    )(x, batch3, *ws)


def _gather_rows(table, src_idx2d, dst_idx2d):
    si = src_idx2d.reshape(-1)
    di = dst_idx2d.reshape(-1)
    return table[si], table[di]


def _scatter_msgs(msgs, dst_idx2d):
    di = dst_idx2d.reshape(-1)
    acc = jax.ops.segment_sum(msgs, di, num_segments=N_PAD)
    return acc, jnp.zeros_like(acc)


def _gvp_w(p):
    out = {'whT': p['wh'].T, 'wswT': p['ws_w'].T, 'wsb': p['ws_b'],
           'wvT': p['wv'].T, 'wsvT': p['wsv_w'].T, 'wsvb': p['wsv_b']}
    return out


def _prep_weights(params):
    w = {}
    w['node'] = dict(_gvp_w(params['node_emb']), lnb=params['node_ln']['b'])
    w['edge'] = dict(_gvp_w(params['edge_emb']), lnb=params['edge_ln']['b'])
    for i in range(2):
        lp = params['layer%d' % i]
        w['layer%d' % i] = {
            'msg': {m: _gvp_w(lp['m%d' % j]) for j, m in
                    ((0, 'm0'), (1, 'm1'), (2, 'm2'))},
            'upd': dict(
                n0w=lp['norm0']['w'], n0b=lp['norm0']['b'],
                n1w=lp['norm1']['w'], n1b=lp['norm1']['b'],
                ff0=_gvp_w(lp['ff0']), ff1=_gvp_w(lp['ff1'])),
        }
    po = params['out']
    w['out'] = {'whT': po['wh'].T, 'wswT': po['ws_w'].T, 'wsb': po['ws_b']}
    return w


def kernel(node_s, node_v, edge_index, edge_s, edge_v, batch, params):
    n = node_s.shape[0]
    e = edge_index.shape[1]
    w = _prep_weights(params)

    epad = E_PAD - e
    src = jnp.pad(edge_index[0], (0, epad)).reshape(-1, 128)
    dst = jnp.pad(edge_index[1], (0, epad),
                  constant_values=N_NODES_C).reshape(-1, 128)
    es_p = jnp.pad(edge_s, ((0, epad), (0, 0)))
    ev_p = jnp.pad(edge_v, ((0, epad), (0, 0)))

    x = _embed_nodes(node_s, node_v, w['node'])
    et = _embed_edges(es_p, ev_p, w['edge'])

    for i in range(2):
        lw = w['layer%d' % i]
        srows, drows = _gather_rows(x, src, dst)
        msgs = _messages(srows, drows, et, lw['msg'])
        a0, a1 = _scatter_msgs(msgs, dst)
        x = _node_update(x, a0[:n], a1[:n], lw['upd'])

    batch3 = batch.reshape(n // BN, 1, BN)
    pooled = _out_pool(x, batch3, w['out'])
    return pooled.reshape(NG)


# speedup vs baseline: 1.8913x; 1.8913x over previous
import functools

import jax
import jax.numpy as jnp
from jax import lax
from jax.experimental import pallas as pl
from jax.experimental.pallas import tpu as pltpu
from jax.experimental.pallas import tpu_sc as plsc

N_NODES_C = 50000
N_EDGES_C = 800000
E_PAD = 819200
N_PAD = 50048
NG = 16
EPS = 1e-8

BN = 2000
BE = 4096


def _fullspec(shape):
    return pl.BlockSpec(shape, lambda i: tuple(0 for _ in shape))


def _r2(b):
    return b.reshape(1, -1)


def _sigmoid(x):
    return jax.nn.sigmoid(x)


def _dot(a, b):
    return jax.lax.dot_general(a, b, (((1,), (0,)), ((), ())),
                               preferred_element_type=jnp.float32)


def _embed_nodes_body(ns, nv, lnb, whT, wswT, wsb, wvT, wsvT, wsvb, out):
    s_ln = jnp.zeros_like(ns[...]) + lnb[...]
    v = nv[...]
    nsq = jnp.maximum(jnp.sum(v * v, axis=1, keepdims=True), EPS)
    v = v / jnp.sqrt(nsq)
    vh = [_dot(v[:, x:x + 1], whT[...]) for x in range(3)]
    vn = jnp.sqrt(jnp.maximum(vh[0] ** 2 + vh[1] ** 2 + vh[2] ** 2, EPS))
    s = _dot(jnp.concatenate([s_ln, vn], axis=1), wswT[...]) + wsb[...]
    gate = _sigmoid(_dot(s, wsvT[...]) + wsvb[...])
    vo = [_dot(vh[x], wvT[...]) * gate for x in range(3)]
    z = jnp.zeros_like(s[:, :2])
    out[...] = jnp.concatenate([s] + vo + [z], axis=1)


def _embed_edges_body(es0, ev0, lnb, whT, wswT, wsb, wvT, wsvT, wsvb, out):
    s_ln = jnp.zeros_like(es0[...]) + lnb[...]
    v = ev0[...]
    nsq = jnp.maximum(jnp.sum(v * v, axis=1, keepdims=True), EPS)
    v = v / jnp.sqrt(nsq)
    vh = [_dot(v[:, x:x + 1], whT[...]) for x in range(3)]
    vn = jnp.sqrt(jnp.maximum(vh[0] ** 2 + vh[1] ** 2 + vh[2] ** 2, EPS))
    s = _dot(jnp.concatenate([s_ln, vn], axis=1), wswT[...]) + wsb[...]
    gate = _sigmoid(_dot(s, wsvT[...]) + wsvb[...])
    vo = [vh[x] * wvT[...] * gate for x in range(3)]
    z = jnp.zeros_like(gate)
    out[...] = jnp.concatenate([s] + vo + [z], axis=1)


def _gvp_block(s_in, v_in, whT, wswT, wsb, wvT, wsvT, wsvb, relu):
    vh = [_dot(v_in[x], whT) for x in range(3)]
    vn = jnp.sqrt(jnp.maximum(vh[0] ** 2 + vh[1] ** 2 + vh[2] ** 2, EPS))
    s = _dot(jnp.concatenate([s_in, vn], axis=1), wswT) + wsb
    gate = _sigmoid(_dot(s, wsvT) + wsvb)
    v = [_dot(vh[x], wvT) * gate for x in range(3)]
    if relu:
        s = jnp.maximum(s, 0.0)
    return s, v


def _messages_body(src, dst, et, *ws):
    (m0whT, m0wswT, m0wsb, m0wvT, m0wsvT, m0wsvb,
     m1whT, m1wswT, m1wsb, m1wvT, m1wsvT, m1wsvb,
     m2whT, m2wswT, m2wsb, m2wvT, m2wsvT, m2wsvb, out) = ws
    S = src[...]
    D = dst[...]
    ET = et[...]
    s_in = jnp.concatenate([S[:, :8], ET[:, :4], D[:, :8]], axis=1)
    v_in = [jnp.concatenate([S[:, 8 + 2 * x:10 + 2 * x],
                             ET[:, 4 + x:5 + x],
                             D[:, 8 + 2 * x:10 + 2 * x]], axis=1)
            for x in range(3)]
    s, v = _gvp_block(s_in, v_in, m0whT[...], m0wswT[...], m0wsb[...],
                      m0wvT[...], m0wsvT[...], m0wsvb[...], True)
    s, v = _gvp_block(s, v, m1whT[...], m1wswT[...], m1wsb[...],
                      m1wvT[...], m1wsvT[...], m1wsvb[...], True)
    s, v = _gvp_block(s, v, m2whT[...], m2wswT[...], m2wsb[...],
                      m2wvT[...], m2wsvT[...], m2wsvb[...], False)
    one = jnp.ones_like(s[:, :1])
    z = jnp.zeros_like(one)
    out[...] = jnp.concatenate([s] + v + [one, z], axis=1)


def _tuple_ln(s, v, w, b):
    mu = jnp.mean(s, axis=1, keepdims=True)
    var = jnp.mean((s - mu) ** 2, axis=1, keepdims=True)
    sn = (s - mu) / jnp.sqrt(var + 1e-5) * w + b
    nsq = jnp.maximum(v[0] ** 2 + v[1] ** 2 + v[2] ** 2, EPS)
    vnorm = jnp.sqrt(jnp.mean(nsq, axis=1, keepdims=True))
    return sn, [v[x] / vnorm for x in range(3)]


def _node_update_body(xin, a0, a1, *ws):
    (n0w, n0b,
     f0whT, f0wswT, f0wsb, f0wvT, f0wsvT, f0wsvb,
     f1whT, f1wswT, f1wsb, f1wvT, f1wsvT, f1wsvb,
     n1w, n1b, out) = ws
    X = xin[...]
    A = a0[...] + a1[...]
    cnt = jnp.maximum(A[:, 14:15], 1.0)
    s = X[:, :8] + A[:, :8] / cnt
    v = [X[:, 8 + 2 * x:10 + 2 * x] + A[:, 8 + 2 * x:10 + 2 * x] / cnt
         for x in range(3)]
    s, v = _tuple_ln(s, v, n0w[...], n0b[...])
    fs, fv = _gvp_block(s, v, f0whT[...], f0wswT[...], f0wsb[...],
                        f0wvT[...], f0wsvT[...], f0wsvb[...], True)
    fs, fv = _gvp_block(fs, fv, f1whT[...], f1wswT[...], f1wsb[...],
                        f1wvT[...], f1wsvT[...], f1wsvb[...], False)
    s = s + fs
    v = [v[x] + fv[x] for x in range(3)]
    s, v = _tuple_ln(s, v, n1w[...], n1b[...])
    z = jnp.zeros_like(s[:, :2])
    out[...] = jnp.concatenate([s] + v + [z], axis=1)


def _out_pool_body(xin, bat, owhT, owswT, owsb, out, acc_s, acc_c):
    i = pl.program_id(0)
    X = xin[...]
    s = X[:, :8]
    v = [X[:, 8 + 2 * x:10 + 2 * x] for x in range(3)]
    vh = [_dot(v[x], owhT[...]) for x in range(3)]
    vn = jnp.sqrt(jnp.maximum(vh[0] ** 2 + vh[1] ** 2 + vh[2] ** 2, EPS))
    o = _dot(jnp.concatenate([s, vn], axis=1), owswT[...]) + owsb[...]
    bb = bat[0, 0, :]
    gid = jax.lax.broadcasted_iota(jnp.int32, (1, NG), 1)
    onehot = (bb[:, None] == gid).astype(jnp.float32)

    @pl.when(i == 0)
    def _():
        acc_s[...] = jnp.zeros_like(acc_s)
        acc_c[...] = jnp.zeros_like(acc_c)

    acc_s[...] += jnp.sum(onehot * o, axis=0, keepdims=True)
    acc_c[...] += jnp.sum(onehot, axis=0, keepdims=True)

    @pl.when(i == pl.num_programs(0) - 1)
    def _():
        out[...] = acc_s[...] / jnp.maximum(acc_c[...], 1.0)


def _embed_nodes(node_s, node_v, p):
    n = node_s.shape[0]
    ws = (_r2(p['lnb']), p['whT'], p['wswT'], _r2(p['wsb']),
          p['wvT'], p['wsvT'], _r2(p['wsvb']))
    grid = n // BN
    specs = [pl.BlockSpec((BN, 1), lambda i: (i, 0)),
             pl.BlockSpec((BN, 3), lambda i: (i, 0))] + \
            [_fullspec(w.shape) for w in ws]
    return pl.pallas_call(
        _embed_nodes_body,
        grid=(grid,),
        in_specs=specs,
        out_specs=pl.BlockSpec((BN, 16), lambda i: (i, 0)),
        out_shape=jax.ShapeDtypeStruct((n, 16), jnp.float32),
    )(node_s, node_v, *ws)


def _embed_edges(edge_s, edge_v, p):
    n = edge_s.shape[0]
    ws = (_r2(p['lnb']), p['whT'], p['wswT'], _r2(p['wsb']),
          p['wvT'], p['wsvT'], _r2(p['wsvb']))
    grid = n // BE
    specs = [pl.BlockSpec((BE, 1), lambda i: (i, 0)),
             pl.BlockSpec((BE, 3), lambda i: (i, 0))] + \
            [_fullspec(w.shape) for w in ws]
    return pl.pallas_call(
        _embed_edges_body,
        grid=(grid,),
        in_specs=specs,
        out_specs=pl.BlockSpec((BE, 8), lambda i: (i, 0)),
        out_shape=jax.ShapeDtypeStruct((n, 8), jnp.float32),
    )(edge_s, edge_v, *ws)


def _messages(src_rows, dst_rows, et, mp):
    n = src_rows.shape[0]
    ws = []
    for m in ('m0', 'm1', 'm2'):
        q = mp[m]
        ws += [q['whT'], q['wswT'], _r2(q['wsb']), q['wvT'], q['wsvT'],
               _r2(q['wsvb'])]
    grid = n // BE
    specs = [pl.BlockSpec((BE, 16), lambda i: (i, 0)),
             pl.BlockSpec((BE, 16), lambda i: (i, 0)),
             pl.BlockSpec((BE, 8), lambda i: (i, 0))] + \
            [_fullspec(w.shape) for w in ws]
    return pl.pallas_call(
        _messages_body,
        grid=(grid,),
        in_specs=specs,
        out_specs=pl.BlockSpec((BE, 16), lambda i: (i, 0)),
        out_shape=jax.ShapeDtypeStruct((n, 16), jnp.float32),
    )(src_rows, dst_rows, et, *ws)


def _node_update(x, a0, a1, lp):
    n = x.shape[0]
    ws = [_r2(lp['n0w']), _r2(lp['n0b'])]
    for m in ('ff0', 'ff1'):
        q = lp[m]
        ws += [q['whT'], q['wswT'], _r2(q['wsb']), q['wvT'], q['wsvT'],
               _r2(q['wsvb'])]
    ws += [_r2(lp['n1w']), _r2(lp['n1b'])]
    grid = n // BN
    specs = [pl.BlockSpec((BN, 16), lambda i: (i, 0))] * 3 + \
            [_fullspec(w.shape) for w in ws]
    return pl.pallas_call(
        _node_update_body,
        grid=(grid,),
        in_specs=specs,
        out_specs=pl.BlockSpec((BN, 16), lambda i: (i, 0)),
        out_shape=jax.ShapeDtypeStruct((n, 16), jnp.float32),
    )(x, a0, a1, *ws)


def _out_pool(x, batch3, op):
    n = x.shape[0]
    ws = (op['whT'], op['wswT'], _r2(op['wsb']))
    grid = n // BN
    specs = [pl.BlockSpec((BN, 16), lambda i: (i, 0)),
             pl.BlockSpec((1, 1, BN), lambda i: (i, 0, 0))] + \
            [_fullspec(w.shape) for w in ws]
    return pl.pallas_call(
        _out_pool_body,
        grid=(grid,),
        in_specs=specs,
        out_specs=_fullspec((1, NG)),
        out_shape=jax.ShapeDtypeStruct((1, NG), jnp.float32),
        scratch_shapes=[pltpu.VMEM((1, NG), jnp.float32),
                        pltpu.VMEM((1, NG), jnp.float32)],
    )(x, batch3, *ws)


NW = 32
IDX_ROWS = E_PAD // 128
ROWS_W = IDX_ROWS // NW
CH = 8
N_STRIPE = N_PAD // 16


def _gather_rows(table, src_idx2d, dst_idx2d):
    mesh = plsc.VectorSubcoreMesh(core_axis_name="c", subcore_axis_name="s", num_cores=2)

    @functools.partial(
        pl.kernel, mesh=mesh,
        compiler_params=pltpu.CompilerParams(use_tc_tiling_on_sc=False),
        out_type=(jax.ShapeDtypeStruct((E_PAD, 16), jnp.float32),
                  jax.ShapeDtypeStruct((E_PAD, 16), jnp.float32)),
        scratch_types=[pltpu.VMEM((CH, 128), jnp.int32),
                       pltpu.VMEM((CH, 128), jnp.int32),
                       pltpu.VMEM((CH * 128, 16), jnp.float32),
                       pltpu.VMEM((CH * 128, 16), jnp.float32),
                       pltpu.SemaphoreType.DMA],
    )
    def k(tab, sidx, didx, so, do, sv, dv, srow, drow, sem):
        wid = lax.axis_index("s") * 2 + lax.axis_index("c")
        base = wid * ROWS_W

        def body(t, carry):
            r0 = base + t * CH
            pltpu.sync_copy(sidx.at[pl.ds(r0, CH)], sv)
            pltpu.sync_copy(didx.at[pl.ds(r0, CH)], dv)
            cps = []
            for j in range(CH):
                cps.append(pltpu.async_copy(
                    tab.at[sv.at[j]], srow.at[pl.ds(j * 128, 128)], sem))
                cps.append(pltpu.async_copy(
                    tab.at[dv.at[j]], drow.at[pl.ds(j * 128, 128)], sem))
            for cp in cps:
                cp.wait()
            pltpu.sync_copy(srow, so.at[pl.ds(r0 * 128, CH * 128)])
            pltpu.sync_copy(drow, do.at[pl.ds(r0 * 128, CH * 128)])
            return carry

        lax.fori_loop(0, ROWS_W // CH, body, 0)

    return k(table, src_idx2d, dst_idx2d)


def _scatter_msgs(msgs, dst_idx2d, zeros_pad):
    mesh = plsc.VectorSubcoreMesh(core_axis_name="c", subcore_axis_name="s", num_cores=2)

    @functools.partial(
        pl.kernel, mesh=mesh,
        compiler_params=pltpu.CompilerParams(use_tc_tiling_on_sc=False),
        out_type=jax.ShapeDtypeStruct((2, N_PAD, 16), jnp.float32),
        scratch_types=[pltpu.VMEM((CH, 128), jnp.int32),
                       pltpu.VMEM((CH * 128, 16), jnp.float32),
                       pltpu.VMEM_SHARED((N_PAD, 16), jnp.float32)],
    )
    def k(m, didx, zeros, out, dv, mv, acc):
        cid = lax.axis_index("c")
        sid = lax.axis_index("s")
        wid = sid * 2 + cid
        base = wid * ROWS_W
        pltpu.sync_copy(zeros.at[pl.ds(sid * N_STRIPE, N_STRIPE)],
                        acc.at[pl.ds(sid * N_STRIPE, N_STRIPE)])
        plsc.subcore_barrier()

        def body(t, carry):
            r0 = base + t * CH
            pltpu.sync_copy(didx.at[pl.ds(r0, CH)], dv)
            pltpu.sync_copy(m.at[pl.ds(r0 * 128, CH * 128)], mv)
            for j in range(CH):
                pltpu.sync_copy(mv.at[pl.ds(j * 128, 128)],
                                acc.at[dv.at[j]], add=True)
            return carry

        lax.fori_loop(0, ROWS_W // CH, body, 0)
        plsc.subcore_barrier()
        pltpu.sync_copy(acc.at[pl.ds(sid * N_STRIPE, N_STRIPE)],
                        out.at[cid, pl.ds(sid * N_STRIPE, N_STRIPE)])

    return k(msgs, dst_idx2d, zeros_pad)


def _gvp_w(p):
    out = {'whT': p['wh'].T, 'wswT': p['ws_w'].T, 'wsb': p['ws_b'],
           'wvT': p['wv'].T, 'wsvT': p['wsv_w'].T, 'wsvb': p['wsv_b']}
    return out


def _prep_weights(params):
    w = {}
    w['node'] = dict(_gvp_w(params['node_emb']), lnb=params['node_ln']['b'])
    w['edge'] = dict(_gvp_w(params['edge_emb']), lnb=params['edge_ln']['b'])
    for i in range(2):
        lp = params['layer%d' % i]
        w['layer%d' % i] = {
            'msg': {m: _gvp_w(lp['m%d' % j]) for j, m in
                    ((0, 'm0'), (1, 'm1'), (2, 'm2'))},
            'upd': dict(
                n0w=lp['norm0']['w'], n0b=lp['norm0']['b'],
                n1w=lp['norm1']['w'], n1b=lp['norm1']['b'],
                ff0=_gvp_w(lp['ff0']), ff1=_gvp_w(lp['ff1'])),
        }
    po = params['out']
    w['out'] = {'whT': po['wh'].T, 'wswT': po['ws_w'].T, 'wsb': po['ws_b']}
    return w


def kernel(node_s, node_v, edge_index, edge_s, edge_v, batch, params):
    n = node_s.shape[0]
    e = edge_index.shape[1]
    w = _prep_weights(params)

    epad = E_PAD - e
    src = jnp.pad(edge_index[0], (0, epad)).reshape(-1, 128)
    dst_g = jnp.pad(edge_index[1], (0, epad)).reshape(-1, 128)
    dst_s = jnp.pad(edge_index[1], (0, epad),
                    constant_values=N_NODES_C).reshape(-1, 128)
    es_p = jnp.pad(edge_s, ((0, epad), (0, 0)))
    ev_p = jnp.pad(edge_v, ((0, epad), (0, 0)))
    zeros_pad = jnp.zeros((N_PAD, 16), jnp.float32)

    x = _embed_nodes(node_s, node_v, w['node'])
    et = _embed_edges(es_p, ev_p, w['edge'])

    for i in range(2):
        lw = w['layer%d' % i]
        srows, drows = _gather_rows(x, src, dst_g)
        msgs = _messages(srows, drows, et, lw['msg'])
        acc = _scatter_msgs(msgs, dst_s, zeros_pad)
        x = _node_update(x, acc[0, :n], acc[1, :n], lw['upd'])

    batch3 = batch.reshape(n // BN, 1, BN)
    pooled = _out_pool(x, batch3, w['out'])
    return pooled.reshape(NG)


# speedup vs baseline: 2.0571x; 1.0877x over previous
import functools

import jax
import jax.numpy as jnp
from jax import lax
from jax.experimental import pallas as pl
from jax.experimental.pallas import tpu as pltpu
from jax.experimental.pallas import tpu_sc as plsc

N_NODES_C = 50000
N_EDGES_C = 800000
E_PAD = 819200
N_PAD = 50176
NG = 16
EPS = 1e-8

BN = 3136
BER = 256
BNR = 224


def _fullspec(shape):
    return pl.BlockSpec(shape, lambda i: tuple(0 for _ in shape))


def _r2(b):
    return b.reshape(1, -1)


def _sigmoid(x):
    return jax.nn.sigmoid(x)


def _dot(a, b):
    return jax.lax.dot_general(a, b, (((1,), (0,)), ((), ())),
                               preferred_element_type=jnp.float32)


def _embed_nodes_body(ns, nv, lnb, whT, wswT, wsb, wvT, wsvT, wsvb, out):
    s_ln = jnp.zeros_like(ns[...]) + lnb[...]
    v = nv[...]
    nsq = jnp.maximum(jnp.sum(v * v, axis=1, keepdims=True), EPS)
    v = v / jnp.sqrt(nsq)
    vh = [_dot(v[:, x:x + 1], whT[...]) for x in range(3)]
    vn = jnp.sqrt(jnp.maximum(vh[0] ** 2 + vh[1] ** 2 + vh[2] ** 2, EPS))
    s = _dot(jnp.concatenate([s_ln, vn], axis=1), wswT[...]) + wsb[...]
    gate = _sigmoid(_dot(s, wsvT[...]) + wsvb[...])
    vo = [_dot(vh[x], wvT[...]) * gate for x in range(3)]
    z = jnp.zeros_like(s[:, :2])
    out[...] = jnp.concatenate([s] + vo + [z], axis=1)


def _gvp_block(s_in, v_in, whT, wswT, wsb, wvT, wsvT, wsvb, relu):
    vh = [_dot(v_in[x], whT) for x in range(3)]
    vn = jnp.sqrt(jnp.maximum(vh[0] ** 2 + vh[1] ** 2 + vh[2] ** 2, EPS))
    s = _dot(jnp.concatenate([s_in, vn], axis=1), wswT) + wsb
    gate = _sigmoid(_dot(s, wsvT) + wsvb)
    v = [_dot(vh[x], wvT) * gate for x in range(3)]
    if relu:
        s = jnp.maximum(s, 0.0)
    return s, v


def _messages_body(src, dst, raw, *ws):
    (elnb, ewhT, ewswT, ewsb, ewvT, ewsvT, ewsvb,
     m0whT, m0wswT, m0wsb, m0wvT, m0wsvT, m0wsvb,
     m1whT, m1wswT, m1wsb, m1wvT, m1wsvT, m1wsvb,
     m2whT, m2wswT, m2wsb, m2wvT, m2wsvT, m2wsvb, out) = ws
    R = src.shape[0]
    Sb, Db, Rb = src[...], dst[...], raw[...]
    S = jnp.concatenate([Sb[:, 16 * k:16 * k + 16] for k in range(8)], axis=0)
    D = jnp.concatenate([Db[:, 16 * k:16 * k + 16] for k in range(8)], axis=0)
    RW = jnp.concatenate([Rb[:, 16 * k:16 * k + 16] for k in range(8)], axis=0)
    es0 = jnp.zeros_like(RW[:, 0:1]) + elnb[...]
    evr = RW[:, 1:4]
    nsq = jnp.maximum(jnp.sum(evr * evr, axis=1, keepdims=True), EPS)
    evr = evr / jnp.sqrt(nsq)
    evh = [evr[:, x:x + 1] * ewhT[...] for x in range(3)]
    evn = jnp.sqrt(jnp.maximum(evh[0] ** 2 + evh[1] ** 2 + evh[2] ** 2, EPS))
    es = _dot(jnp.concatenate([es0, evn], axis=1), ewswT[...]) + ewsb[...]
    egate = _sigmoid(_dot(es, ewsvT[...]) + ewsvb[...])
    ev = [evh[x] * ewvT[...] * egate for x in range(3)]
    s_in = jnp.concatenate([S[:, :8], es, D[:, :8]], axis=1)
    v_in = [jnp.concatenate([S[:, 8 + 2 * x:10 + 2 * x],
                             ev[x],
                             D[:, 8 + 2 * x:10 + 2 * x]], axis=1)
            for x in range(3)]
    s, v = _gvp_block(s_in, v_in, m0whT[...], m0wswT[...], m0wsb[...],
                      m0wvT[...], m0wsvT[...], m0wsvb[...], True)
    s, v = _gvp_block(s, v, m1whT[...], m1wswT[...], m1wsb[...],
                      m1wvT[...], m1wsvT[...], m1wsvb[...], True)
    s, v = _gvp_block(s, v, m2whT[...], m2wswT[...], m2wsb[...],
                      m2wvT[...], m2wsvT[...], m2wsvb[...], False)
    one = jnp.ones_like(s[:, :1])
    z = jnp.zeros_like(one)
    M = jnp.concatenate([s] + v + [one, z], axis=1)
    out[...] = jnp.concatenate([M[k * R:(k + 1) * R] for k in range(8)],
                               axis=1)


def _tuple_ln(s, v, w, b):
    mu = jnp.mean(s, axis=1, keepdims=True)
    var = jnp.mean((s - mu) ** 2, axis=1, keepdims=True)
    sn = (s - mu) / jnp.sqrt(var + 1e-5) * w + b
    nsq = jnp.maximum(v[0] ** 2 + v[1] ** 2 + v[2] ** 2, EPS)
    vnorm = jnp.sqrt(jnp.mean(nsq, axis=1, keepdims=True))
    return sn, [v[x] / vnorm for x in range(3)]


def _node_update_body(xin, a0, a1, *ws):
    (n0w, n0b,
     f0whT, f0wswT, f0wsb, f0wvT, f0wsvT, f0wsvb,
     f1whT, f1wswT, f1wsb, f1wvT, f1wsvT, f1wsvb,
     n1w, n1b, out) = ws
    R = xin.shape[0]
    Xb = xin[...]
    Ab = a0[...] + a1[...]
    X = jnp.concatenate([Xb[:, 16 * k:16 * k + 16] for k in range(8)], axis=0)
    A = jnp.concatenate([Ab[:, 16 * k:16 * k + 16] for k in range(8)], axis=0)
    cnt = jnp.maximum(A[:, 14:15], 1.0)
    s = X[:, :8] + A[:, :8] / cnt
    v = [X[:, 8 + 2 * x:10 + 2 * x] + A[:, 8 + 2 * x:10 + 2 * x] / cnt
         for x in range(3)]
    s, v = _tuple_ln(s, v, n0w[...], n0b[...])
    fs, fv = _gvp_block(s, v, f0whT[...], f0wswT[...], f0wsb[...],
                        f0wvT[...], f0wsvT[...], f0wsvb[...], True)
    fs, fv = _gvp_block(fs, fv, f1whT[...], f1wswT[...], f1wsb[...],
                        f1wvT[...], f1wsvT[...], f1wsvb[...], False)
    s = s + fs
    v = [v[x] + fv[x] for x in range(3)]
    s, v = _tuple_ln(s, v, n1w[...], n1b[...])
    z = jnp.zeros_like(s[:, :2])
    Y = jnp.concatenate([s] + v + [z], axis=1)
    out[...] = jnp.concatenate([Y[k * R:(k + 1) * R] for k in range(8)],
                               axis=1)


def _out_pool_body(xin, bat, owhT, owswT, owsb, out, acc_s, acc_c):
    i = pl.program_id(0)
    Xb = xin[...]
    X = jnp.concatenate([Xb[:, 16 * k:16 * k + 16] for k in range(8)], axis=0)
    bb = bat[...]
    bs = jnp.concatenate([bb[:, k:k + 1] for k in range(8)], axis=0)
    s = X[:, :8]
    v = [X[:, 8 + 2 * x:10 + 2 * x] for x in range(3)]
    vh = [_dot(v[x], owhT[...]) for x in range(3)]
    vn = jnp.sqrt(jnp.maximum(vh[0] ** 2 + vh[1] ** 2 + vh[2] ** 2, EPS))
    o = _dot(jnp.concatenate([s, vn], axis=1), owswT[...]) + owsb[...]
    gid = jax.lax.broadcasted_iota(jnp.int32, (1, NG), 1)
    onehot = (bs == gid).astype(jnp.float32)

    @pl.when(i == 0)
    def _():
        acc_s[...] = jnp.zeros_like(acc_s)
        acc_c[...] = jnp.zeros_like(acc_c)

    acc_s[...] += jnp.sum(onehot * o, axis=0, keepdims=True)
    acc_c[...] += jnp.sum(onehot, axis=0, keepdims=True)

    @pl.when(i == pl.num_programs(0) - 1)
    def _():
        out[...] = acc_s[...] / jnp.maximum(acc_c[...], 1.0)


def _embed_nodes(node_s, node_v, p):
    n = node_s.shape[0]
    ws = (_r2(p['lnb']), p['whT'], p['wswT'], _r2(p['wsb']),
          p['wvT'], p['wsvT'], _r2(p['wsvb']))
    grid = n // BN
    specs = [pl.BlockSpec((BN, 1), lambda i: (i, 0)),
             pl.BlockSpec((BN, 3), lambda i: (i, 0))] + \
            [_fullspec(w.shape) for w in ws]
    return pl.pallas_call(
        _embed_nodes_body,
        grid=(grid,),
        in_specs=specs,
        out_specs=pl.BlockSpec((BN, 16), lambda i: (i, 0)),
        out_shape=jax.ShapeDtypeStruct((n, 16), jnp.float32),
    )(node_s, node_v, *ws)


def _messages(src128, dst128, raw16, we, mp):
    n = src128.shape[0]
    ws = [_r2(we['lnb']), we['whT'], we['wswT'], _r2(we['wsb']),
          we['wvT'], we['wsvT'], _r2(we['wsvb'])]
    for m in ('m0', 'm1', 'm2'):
        q = mp[m]
        ws += [q['whT'], q['wswT'], _r2(q['wsb']), q['wvT'], q['wsvT'],
               _r2(q['wsvb'])]
    grid = n // BER
    specs = [pl.BlockSpec((BER, 128), lambda i: (i, 0))] * 3 +             [_fullspec(w.shape) for w in ws]
    return pl.pallas_call(
        _messages_body,
        grid=(grid,),
        in_specs=specs,
        out_specs=pl.BlockSpec((BER, 128), lambda i: (i, 0)),
        out_shape=jax.ShapeDtypeStruct((n, 128), jnp.float32),
    )(src128, dst128, raw16, *ws)


def _node_update(x128, a0, a1, lp):
    n = x128.shape[0]
    ws = [_r2(lp['n0w']), _r2(lp['n0b'])]
    for m in ('ff0', 'ff1'):
        q = lp[m]
        ws += [q['whT'], q['wswT'], _r2(q['wsb']), q['wvT'], q['wsvT'],
               _r2(q['wsvb'])]
    ws += [_r2(lp['n1w']), _r2(lp['n1b'])]
    grid = n // BNR
    specs = [pl.BlockSpec((BNR, 128), lambda i: (i, 0))] * 3 +             [_fullspec(w.shape) for w in ws]
    return pl.pallas_call(
        _node_update_body,
        grid=(grid,),
        in_specs=specs,
        out_specs=pl.BlockSpec((BNR, 128), lambda i: (i, 0)),
        out_shape=jax.ShapeDtypeStruct((n, 128), jnp.float32),
    )(x128, a0, a1, *ws)


def _out_pool(x128, batch8, op):
    n = x128.shape[0]
    ws = (op['whT'], op['wswT'], _r2(op['wsb']))
    grid = n // BNR
    specs = [pl.BlockSpec((BNR, 128), lambda i: (i, 0)),
             pl.BlockSpec((BNR, 8), lambda i: (i, 0))] +             [_fullspec(w.shape) for w in ws]
    return pl.pallas_call(
        _out_pool_body,
        grid=(grid,),
        in_specs=specs,
        out_specs=_fullspec((1, NG)),
        out_shape=jax.ShapeDtypeStruct((1, NG), jnp.float32),
        scratch_shapes=[pltpu.VMEM((1, NG), jnp.float32),
                        pltpu.VMEM((1, NG), jnp.float32)],
    )(x128, batch8, *ws)


NW = 32
IDX_ROWS = E_PAD // 128
ROWS_W = IDX_ROWS // NW
CH = 8
N_STRIPE = N_PAD // 16


def _gather_rows(table, src_idx2d, dst_idx2d):
    mesh = plsc.VectorSubcoreMesh(core_axis_name="c", subcore_axis_name="s", num_cores=2)

    @functools.partial(
        pl.kernel, mesh=mesh,
        compiler_params=pltpu.CompilerParams(use_tc_tiling_on_sc=False),
        out_type=(jax.ShapeDtypeStruct((E_PAD, 16), jnp.float32),
                  jax.ShapeDtypeStruct((E_PAD, 16), jnp.float32)),
        scratch_types=[pltpu.VMEM((CH, 128), jnp.int32),
                       pltpu.VMEM((CH, 128), jnp.int32),
                       pltpu.VMEM((CH * 128, 16), jnp.float32),
                       pltpu.VMEM((CH * 128, 16), jnp.float32),
                       pltpu.SemaphoreType.DMA],
    )
    def k(tab, sidx, didx, so, do, sv, dv, srow, drow, sem):
        wid = lax.axis_index("s") * 2 + lax.axis_index("c")
        base = wid * ROWS_W

        def body(t, carry):
            r0 = base + t * CH
            pltpu.sync_copy(sidx.at[pl.ds(r0, CH)], sv)
            pltpu.sync_copy(didx.at[pl.ds(r0, CH)], dv)
            cps = []
            for j in range(CH):
                cps.append(pltpu.async_copy(
                    tab.at[sv.at[j]], srow.at[pl.ds(j * 128, 128)], sem))
                cps.append(pltpu.async_copy(
                    tab.at[dv.at[j]], drow.at[pl.ds(j * 128, 128)], sem))
            for cp in cps:
                cp.wait()
            pltpu.sync_copy(srow, so.at[pl.ds(r0 * 128, CH * 128)])
            pltpu.sync_copy(drow, do.at[pl.ds(r0 * 128, CH * 128)])
            return carry

        lax.fori_loop(0, ROWS_W // CH, body, 0)

    return k(table, src_idx2d, dst_idx2d)


def _scatter_msgs(msgs, dst_idx2d, zeros_pad):
    mesh = plsc.VectorSubcoreMesh(core_axis_name="c", subcore_axis_name="s", num_cores=2)

    @functools.partial(
        pl.kernel, mesh=mesh,
        compiler_params=pltpu.CompilerParams(use_tc_tiling_on_sc=False),
        out_type=jax.ShapeDtypeStruct((2, N_PAD, 16), jnp.float32),
        scratch_types=[pltpu.VMEM((CH, 128), jnp.int32),
                       pltpu.VMEM((CH * 128, 16), jnp.float32),
                       pltpu.VMEM_SHARED((N_PAD, 16), jnp.float32)],
    )
    def k(m, didx, zeros, out, dv, mv, acc):
        cid = lax.axis_index("c")
        sid = lax.axis_index("s")
        wid = sid * 2 + cid
        base = wid * ROWS_W
        pltpu.sync_copy(zeros.at[pl.ds(sid * N_STRIPE, N_STRIPE)],
                        acc.at[pl.ds(sid * N_STRIPE, N_STRIPE)])
        plsc.subcore_barrier()

        def body(t, carry):
            r0 = base + t * CH
            pltpu.sync_copy(didx.at[pl.ds(r0, CH)], dv)
            pltpu.sync_copy(m.at[pl.ds(r0 * 128, CH * 128)], mv)
            for j in range(CH):
                pltpu.sync_copy(mv.at[pl.ds(j * 128, 128)],
                                acc.at[dv.at[j]], add=True)
            return carry

        lax.fori_loop(0, ROWS_W // CH, body, 0)
        plsc.subcore_barrier()
        pltpu.sync_copy(acc.at[pl.ds(sid * N_STRIPE, N_STRIPE)],
                        out.at[cid, pl.ds(sid * N_STRIPE, N_STRIPE)])

    return k(msgs, dst_idx2d, zeros_pad)


def _gvp_w(p):
    out = {'whT': p['wh'].T, 'wswT': p['ws_w'].T, 'wsb': p['ws_b'],
           'wvT': p['wv'].T, 'wsvT': p['wsv_w'].T, 'wsvb': p['wsv_b']}
    return out


def _prep_weights(params):
    w = {}
    w['node'] = dict(_gvp_w(params['node_emb']), lnb=params['node_ln']['b'])
    w['edge'] = dict(_gvp_w(params['edge_emb']), lnb=params['edge_ln']['b'])
    for i in range(2):
        lp = params['layer%d' % i]
        w['layer%d' % i] = {
            'msg': {m: _gvp_w(lp['m%d' % j]) for j, m in
                    ((0, 'm0'), (1, 'm1'), (2, 'm2'))},
            'upd': dict(
                n0w=lp['norm0']['w'], n0b=lp['norm0']['b'],
                n1w=lp['norm1']['w'], n1b=lp['norm1']['b'],
                ff0=_gvp_w(lp['ff0']), ff1=_gvp_w(lp['ff1'])),
        }
    po = params['out']
    w['out'] = {'whT': po['wh'].T, 'wswT': po['ws_w'].T, 'wsb': po['ws_b']}
    return w


def kernel(node_s, node_v, edge_index, edge_s, edge_v, batch, params):
    n = node_s.shape[0]
    e = edge_index.shape[1]
    w = _prep_weights(params)

    epad = E_PAD - e
    npad = N_PAD - n
    src = jnp.pad(edge_index[0], (0, epad)).reshape(-1, 128)
    dst_g = jnp.pad(edge_index[1], (0, epad)).reshape(-1, 128)
    dst_s = jnp.pad(edge_index[1], (0, epad),
                    constant_values=n).reshape(-1, 128)
    raw16 = jnp.pad(jnp.concatenate([edge_s, edge_v], axis=1),
                    ((0, epad), (0, 12))).reshape(E_PAD // 8, 128)
    ns_p = jnp.pad(node_s, ((0, npad), (0, 0)))
    nv_p = jnp.pad(node_v, ((0, npad), (0, 0)))
    batch8 = jnp.pad(batch, (0, npad),
                     constant_values=NG).reshape(N_PAD // 8, 8)
    zeros_pad = jnp.zeros((N_PAD, 16), jnp.float32)

    x16 = _embed_nodes(ns_p, nv_p, w['node'])
    x128 = x16.reshape(N_PAD // 8, 128)

    for i in range(2):
        lw = w['layer%d' % i]
        x_sc = x128.reshape(N_PAD, 16)
        srows, drows = _gather_rows(x_sc, src, dst_g)
        s128 = srows.reshape(E_PAD // 8, 128)
        d128 = drows.reshape(E_PAD // 8, 128)
        m128 = _messages(s128, d128, raw16, w['edge'], lw['msg'])
        msgs = m128.reshape(E_PAD, 16)
        acc = _scatter_msgs(msgs, dst_s, zeros_pad)
        acc128 = acc.reshape(2, N_PAD // 8, 128)
        x128 = _node_update(x128, acc128[0], acc128[1], lw['upd'])

    pooled = _out_pool(x128, batch8, w['out'])
    return pooled.reshape(NG)


# speedup vs baseline: 2.1103x; 1.0258x over previous
import functools

import jax
import jax.numpy as jnp
from jax import lax
from jax.experimental import pallas as pl
from jax.experimental.pallas import tpu as pltpu
from jax.experimental.pallas import tpu_sc as plsc

N_NODES_C = 50000
N_EDGES_C = 800000
E_PAD = 819200
N_PAD = 50176
NG = 16
EPS = 1e-8

BN = 3136
BER = 512
BNR = 448


def _fullspec(shape):
    return pl.BlockSpec(shape, lambda i: tuple(0 for _ in shape))


def _r2(b):
    return b.reshape(1, -1)


def _sigmoid(x):
    return 0.5 * jnp.tanh(0.5 * x) + 0.5


def _dot(a, b):
    return jax.lax.dot_general(a, b, (((1,), (0,)), ((), ())),
                               preferred_element_type=jnp.float32)


def _embed_nodes_body(ns, nv, lnb, whT, wswT, wsb, wvT, wsvT, wsvb, out):
    s_ln = jnp.zeros_like(ns[...]) + lnb[...]
    v = nv[...]
    nsq = jnp.maximum(jnp.sum(v * v, axis=1, keepdims=True), EPS)
    v = v / jnp.sqrt(nsq)
    vh = [_dot(v[:, x:x + 1], whT[...]) for x in range(3)]
    vn = jnp.sqrt(jnp.maximum(vh[0] ** 2 + vh[1] ** 2 + vh[2] ** 2, EPS))
    s = _dot(jnp.concatenate([s_ln, vn], axis=1), wswT[...]) + wsb[...]
    gate = _sigmoid(_dot(s, wsvT[...]) + wsvb[...])
    vo = [_dot(vh[x], wvT[...]) * gate for x in range(3)]
    z = jnp.zeros_like(s[:, :2])
    out[...] = jnp.concatenate([s] + vo + [z], axis=1)


def _gvp_block(s_in, v_in, whT, wswT, wsb, wvT, wsvT, wsvb, relu):
    vh = [_dot(v_in[x], whT) for x in range(3)]
    vn = jnp.sqrt(jnp.maximum(vh[0] ** 2 + vh[1] ** 2 + vh[2] ** 2, EPS))
    s = _dot(jnp.concatenate([s_in, vn], axis=1), wswT) + wsb
    gate = _sigmoid(_dot(s, wsvT) + wsvb)
    v = [_dot(vh[x], wvT) * gate for x in range(3)]
    if relu:
        s = jnp.maximum(s, 0.0)
    return s, v


def _messages_body(src, dst, raw, *ws):
    (elnb, ewhT, ewswT, ewsb, ewvT, ewsvT, ewsvb,
     m0whT, m0wswT, m0wsb, m0wvT, m0wsvT, m0wsvb,
     m1whT, m1wswT, m1wsb, m1wvT, m1wsvT, m1wsvb,
     m2whT, m2wswT, m2wsb, m2wvT, m2wsvT, m2wsvb, out) = ws
    R = src.shape[0]
    Sb, Db, Rb = src[...], dst[...], raw[...]
    S = jnp.concatenate([Sb[:, 16 * k:16 * k + 16] for k in range(8)], axis=0)
    D = jnp.concatenate([Db[:, 16 * k:16 * k + 16] for k in range(8)], axis=0)
    RW = jnp.concatenate([Rb[:, 16 * k:16 * k + 16] for k in range(8)], axis=0)
    es0 = jnp.zeros_like(RW[:, 0:1]) + elnb[...]
    evr = RW[:, 1:4]
    nsq = jnp.maximum(jnp.sum(evr * evr, axis=1, keepdims=True), EPS)
    evr = evr / jnp.sqrt(nsq)
    evh = [evr[:, x:x + 1] * ewhT[...] for x in range(3)]
    evn = jnp.sqrt(jnp.maximum(evh[0] ** 2 + evh[1] ** 2 + evh[2] ** 2, EPS))
    es = _dot(jnp.concatenate([es0, evn], axis=1), ewswT[...]) + ewsb[...]
    egate = _sigmoid(_dot(es, ewsvT[...]) + ewsvb[...])
    ev = [evh[x] * ewvT[...] * egate for x in range(3)]
    s_in = jnp.concatenate([S[:, :8], es, D[:, :8]], axis=1)
    v_in = [jnp.concatenate([S[:, 8 + 2 * x:10 + 2 * x],
                             ev[x],
                             D[:, 8 + 2 * x:10 + 2 * x]], axis=1)
            for x in range(3)]
    s, v = _gvp_block(s_in, v_in, m0whT[...], m0wswT[...], m0wsb[...],
                      m0wvT[...], m0wsvT[...], m0wsvb[...], True)
    s, v = _gvp_block(s, v, m1whT[...], m1wswT[...], m1wsb[...],
                      m1wvT[...], m1wsvT[...], m1wsvb[...], True)
    s, v = _gvp_block(s, v, m2whT[...], m2wswT[...], m2wsb[...],
                      m2wvT[...], m2wsvT[...], m2wsvb[...], False)
    one = jnp.ones_like(s[:, :1])
    z = jnp.zeros_like(one)
    M = jnp.concatenate([s] + v + [one, z], axis=1)
    out[...] = jnp.concatenate([M[k * R:(k + 1) * R] for k in range(8)],
                               axis=1)


def _tuple_ln(s, v, w, b):
    mu = jnp.mean(s, axis=1, keepdims=True)
    var = jnp.mean((s - mu) ** 2, axis=1, keepdims=True)
    sn = (s - mu) / jnp.sqrt(var + 1e-5) * w + b
    nsq = jnp.maximum(v[0] ** 2 + v[1] ** 2 + v[2] ** 2, EPS)
    vnorm = jnp.sqrt(jnp.mean(nsq, axis=1, keepdims=True))
    return sn, [v[x] / vnorm for x in range(3)]


def _node_update_body(xin, a0, a1, *ws):
    (n0w, n0b,
     f0whT, f0wswT, f0wsb, f0wvT, f0wsvT, f0wsvb,
     f1whT, f1wswT, f1wsb, f1wvT, f1wsvT, f1wsvb,
     n1w, n1b, out) = ws
    R = xin.shape[0]
    Xb = xin[...]
    Ab = a0[...] + a1[...]
    X = jnp.concatenate([Xb[:, 16 * k:16 * k + 16] for k in range(8)], axis=0)
    A = jnp.concatenate([Ab[:, 16 * k:16 * k + 16] for k in range(8)], axis=0)
    cnt = jnp.maximum(A[:, 14:15], 1.0)
    s = X[:, :8] + A[:, :8] / cnt
    v = [X[:, 8 + 2 * x:10 + 2 * x] + A[:, 8 + 2 * x:10 + 2 * x] / cnt
         for x in range(3)]
    s, v = _tuple_ln(s, v, n0w[...], n0b[...])
    fs, fv = _gvp_block(s, v, f0whT[...], f0wswT[...], f0wsb[...],
                        f0wvT[...], f0wsvT[...], f0wsvb[...], True)
    fs, fv = _gvp_block(fs, fv, f1whT[...], f1wswT[...], f1wsb[...],
                        f1wvT[...], f1wsvT[...], f1wsvb[...], False)
    s = s + fs
    v = [v[x] + fv[x] for x in range(3)]
    s, v = _tuple_ln(s, v, n1w[...], n1b[...])
    z = jnp.zeros_like(s[:, :2])
    Y = jnp.concatenate([s] + v + [z], axis=1)
    out[...] = jnp.concatenate([Y[k * R:(k + 1) * R] for k in range(8)],
                               axis=1)


def _out_pool_body(xin, bat, owhT, owswT, owsb, out, acc_s, acc_c):
    i = pl.program_id(0)
    Xb = xin[...]
    X = jnp.concatenate([Xb[:, 16 * k:16 * k + 16] for k in range(8)], axis=0)
    bb = bat[...]
    bs = jnp.concatenate([bb[:, k:k + 1] for k in range(8)], axis=0)
    s = X[:, :8]
    v = [X[:, 8 + 2 * x:10 + 2 * x] for x in range(3)]
    vh = [_dot(v[x], owhT[...]) for x in range(3)]
    vn = jnp.sqrt(jnp.maximum(vh[0] ** 2 + vh[1] ** 2 + vh[2] ** 2, EPS))
    o = _dot(jnp.concatenate([s, vn], axis=1), owswT[...]) + owsb[...]
    gid = jax.lax.broadcasted_iota(jnp.int32, (1, NG), 1)
    onehot = (bs == gid).astype(jnp.float32)

    @pl.when(i == 0)
    def _():
        acc_s[...] = jnp.zeros_like(acc_s)
        acc_c[...] = jnp.zeros_like(acc_c)

    acc_s[...] += jnp.sum(onehot * o, axis=0, keepdims=True)
    acc_c[...] += jnp.sum(onehot, axis=0, keepdims=True)

    @pl.when(i == pl.num_programs(0) - 1)
    def _():
        out[...] = acc_s[...] / jnp.maximum(acc_c[...], 1.0)


def _embed_nodes(node_s, node_v, p):
    n = node_s.shape[0]
    ws = (_r2(p['lnb']), p['whT'], p['wswT'], _r2(p['wsb']),
          p['wvT'], p['wsvT'], _r2(p['wsvb']))
    grid = n // BN
    specs = [pl.BlockSpec((BN, 1), lambda i: (i, 0)),
             pl.BlockSpec((BN, 3), lambda i: (i, 0))] + \
            [_fullspec(w.shape) for w in ws]
    return pl.pallas_call(
        _embed_nodes_body,
        grid=(grid,),
        in_specs=specs,
        out_specs=pl.BlockSpec((BN, 16), lambda i: (i, 0)),
        out_shape=jax.ShapeDtypeStruct((n, 16), jnp.float32),
    )(node_s, node_v, *ws)


def _messages(src128, dst128, raw16, we, mp):
    n = src128.shape[0]
    ws = [_r2(we['lnb']), we['whT'], we['wswT'], _r2(we['wsb']),
          we['wvT'], we['wsvT'], _r2(we['wsvb'])]
    for m in ('m0', 'm1', 'm2'):
        q = mp[m]
        ws += [q['whT'], q['wswT'], _r2(q['wsb']), q['wvT'], q['wsvT'],
               _r2(q['wsvb'])]
    grid = n // BER
    specs = [pl.BlockSpec((BER, 128), lambda i: (i, 0))] * 3 +             [_fullspec(w.shape) for w in ws]
    return pl.pallas_call(
        _messages_body,
        grid=(grid,),
        in_specs=specs,
        out_specs=pl.BlockSpec((BER, 128), lambda i: (i, 0)),
        out_shape=jax.ShapeDtypeStruct((n, 128), jnp.float32),
    )(src128, dst128, raw16, *ws)


def _node_update(x128, a0, a1, lp):
    n = x128.shape[0]
    ws = [_r2(lp['n0w']), _r2(lp['n0b'])]
    for m in ('ff0', 'ff1'):
        q = lp[m]
        ws += [q['whT'], q['wswT'], _r2(q['wsb']), q['wvT'], q['wsvT'],
               _r2(q['wsvb'])]
    ws += [_r2(lp['n1w']), _r2(lp['n1b'])]
    grid = n // BNR
    specs = [pl.BlockSpec((BNR, 128), lambda i: (i, 0))] * 3 +             [_fullspec(w.shape) for w in ws]
    return pl.pallas_call(
        _node_update_body,
        grid=(grid,),
        in_specs=specs,
        out_specs=pl.BlockSpec((BNR, 128), lambda i: (i, 0)),
        out_shape=jax.ShapeDtypeStruct((n, 128), jnp.float32),
    )(x128, a0, a1, *ws)


def _out_pool(x128, batch8, op):
    n = x128.shape[0]
    ws = (op['whT'], op['wswT'], _r2(op['wsb']))
    grid = n // BNR
    specs = [pl.BlockSpec((BNR, 128), lambda i: (i, 0)),
             pl.BlockSpec((BNR, 8), lambda i: (i, 0))] +             [_fullspec(w.shape) for w in ws]
    return pl.pallas_call(
        _out_pool_body,
        grid=(grid,),
        in_specs=specs,
        out_specs=_fullspec((1, NG)),
        out_shape=jax.ShapeDtypeStruct((1, NG), jnp.float32),
        scratch_shapes=[pltpu.VMEM((1, NG), jnp.float32),
                        pltpu.VMEM((1, NG), jnp.float32)],
    )(x128, batch8, *ws)


NW = 32
IDX_ROWS = E_PAD // 128
ROWS_W = IDX_ROWS // NW
CH = 8
N_STRIPE = N_PAD // 16


def _gather_rows(table, src_idx2d, dst_idx2d):
    mesh = plsc.VectorSubcoreMesh(core_axis_name="c", subcore_axis_name="s", num_cores=2)

    @functools.partial(
        pl.kernel, mesh=mesh,
        compiler_params=pltpu.CompilerParams(use_tc_tiling_on_sc=False),
        out_type=(jax.ShapeDtypeStruct((E_PAD, 16), jnp.float32),
                  jax.ShapeDtypeStruct((E_PAD, 16), jnp.float32)),
        scratch_types=[pltpu.VMEM((CH, 128), jnp.int32),
                       pltpu.VMEM((CH, 128), jnp.int32),
                       pltpu.VMEM((CH * 128, 16), jnp.float32),
                       pltpu.VMEM((CH * 128, 16), jnp.float32),
                       pltpu.SemaphoreType.DMA],
    )
    def k(tab, sidx, didx, so, do, sv, dv, srow, drow, sem):
        wid = lax.axis_index("s") * 2 + lax.axis_index("c")
        base = wid * ROWS_W

        def body(t, carry):
            r0 = base + t * CH
            pltpu.sync_copy(sidx.at[pl.ds(r0, CH)], sv)
            pltpu.sync_copy(didx.at[pl.ds(r0, CH)], dv)
            cps = []
            for j in range(CH):
                cps.append(pltpu.async_copy(
                    tab.at[sv.at[j]], srow.at[pl.ds(j * 128, 128)], sem))
                cps.append(pltpu.async_copy(
                    tab.at[dv.at[j]], drow.at[pl.ds(j * 128, 128)], sem))
            for cp in cps:
                cp.wait()
            pltpu.sync_copy(srow, so.at[pl.ds(r0 * 128, CH * 128)])
            pltpu.sync_copy(drow, do.at[pl.ds(r0 * 128, CH * 128)])
            return carry

        lax.fori_loop(0, ROWS_W // CH, body, 0)

    return k(table, src_idx2d, dst_idx2d)


def _scatter_msgs(msgs, dst_idx2d, zeros_pad):
    mesh = plsc.VectorSubcoreMesh(core_axis_name="c", subcore_axis_name="s", num_cores=2)

    @functools.partial(
        pl.kernel, mesh=mesh,
        compiler_params=pltpu.CompilerParams(use_tc_tiling_on_sc=False),
        out_type=jax.ShapeDtypeStruct((2, N_PAD, 16), jnp.float32),
        scratch_types=[pltpu.VMEM((CH, 128), jnp.int32),
                       pltpu.VMEM((CH * 128, 16), jnp.float32),
                       pltpu.VMEM_SHARED((N_PAD, 16), jnp.float32)],
    )
    def k(m, didx, zeros, out, dv, mv, acc):
        cid = lax.axis_index("c")
        sid = lax.axis_index("s")
        wid = sid * 2 + cid
        base = wid * ROWS_W
        pltpu.sync_copy(zeros.at[pl.ds(sid * N_STRIPE, N_STRIPE)],
                        acc.at[pl.ds(sid * N_STRIPE, N_STRIPE)])
        plsc.subcore_barrier()

        def body(t, carry):
            r0 = base + t * CH
            pltpu.sync_copy(didx.at[pl.ds(r0, CH)], dv)
            pltpu.sync_copy(m.at[pl.ds(r0 * 128, CH * 128)], mv)
            for j in range(CH):
                pltpu.sync_copy(mv.at[pl.ds(j * 128, 128)],
                                acc.at[dv.at[j]], add=True)
            return carry

        lax.fori_loop(0, ROWS_W // CH, body, 0)
        plsc.subcore_barrier()
        pltpu.sync_copy(acc.at[pl.ds(sid * N_STRIPE, N_STRIPE)],
                        out.at[cid, pl.ds(sid * N_STRIPE, N_STRIPE)])

    return k(msgs, dst_idx2d, zeros_pad)


def _gvp_w(p):
    out = {'whT': p['wh'].T, 'wswT': p['ws_w'].T, 'wsb': p['ws_b'],
           'wvT': p['wv'].T, 'wsvT': p['wsv_w'].T, 'wsvb': p['wsv_b']}
    return out


def _prep_weights(params):
    w = {}
    w['node'] = dict(_gvp_w(params['node_emb']), lnb=params['node_ln']['b'])
    w['edge'] = dict(_gvp_w(params['edge_emb']), lnb=params['edge_ln']['b'])
    for i in range(2):
        lp = params['layer%d' % i]
        w['layer%d' % i] = {
            'msg': {m: _gvp_w(lp['m%d' % j]) for j, m in
                    ((0, 'm0'), (1, 'm1'), (2, 'm2'))},
            'upd': dict(
                n0w=lp['norm0']['w'], n0b=lp['norm0']['b'],
                n1w=lp['norm1']['w'], n1b=lp['norm1']['b'],
                ff0=_gvp_w(lp['ff0']), ff1=_gvp_w(lp['ff1'])),
        }
    po = params['out']
    w['out'] = {'whT': po['wh'].T, 'wswT': po['ws_w'].T, 'wsb': po['ws_b']}
    return w


def kernel(node_s, node_v, edge_index, edge_s, edge_v, batch, params):
    n = node_s.shape[0]
    e = edge_index.shape[1]
    w = _prep_weights(params)

    epad = E_PAD - e
    npad = N_PAD - n
    src = jnp.pad(edge_index[0], (0, epad)).reshape(-1, 128)
    dst_g = jnp.pad(edge_index[1], (0, epad)).reshape(-1, 128)
    dst_s = jnp.pad(edge_index[1], (0, epad),
                    constant_values=n).reshape(-1, 128)
    raw16 = jnp.pad(jnp.concatenate([edge_s, edge_v], axis=1),
                    ((0, epad), (0, 12))).reshape(E_PAD // 8, 128)
    ns_p = jnp.pad(node_s, ((0, npad), (0, 0)))
    nv_p = jnp.pad(node_v, ((0, npad), (0, 0)))
    batch8 = jnp.pad(batch, (0, npad),
                     constant_values=NG).reshape(N_PAD // 8, 8)
    zeros_pad = jnp.zeros((N_PAD, 16), jnp.float32)

    x16 = _embed_nodes(ns_p, nv_p, w['node'])
    x128 = x16.reshape(N_PAD // 8, 128)

    for i in range(2):
        lw = w['layer%d' % i]
        x_sc = x128.reshape(N_PAD, 16)
        srows, drows = _gather_rows(x_sc, src, dst_g)
        s128 = srows.reshape(E_PAD // 8, 128)
        d128 = drows.reshape(E_PAD // 8, 128)
        m128 = _messages(s128, d128, raw16, w['edge'], lw['msg'])
        msgs = m128.reshape(E_PAD, 16)
        acc = _scatter_msgs(msgs, dst_s, zeros_pad)
        acc128 = acc.reshape(2, N_PAD // 8, 128)
        x128 = _node_update(x128, acc128[0], acc128[1], lw['upd'])

    pooled = _out_pool(x128, batch8, w['out'])
    return pooled.reshape(NG)
